# Initial kernel scaffold; baseline (speedup 1.0000x reference)
import jax
import jax.numpy as jnp
from jax.experimental import pallas as pl


def kernel(x, edge_index, u, edge_weight, batch, batch_size, W1, b1, W2, b2, Wh1, bh1, Wh2, bh2):
    raise NotImplementedError("write your pallas kernel here")


# speedup vs baseline: 7.2397x; 7.2397x over previous
import jax
import jax.numpy as jnp
from jax import lax
from jax.experimental import pallas as pl
from jax.experimental.pallas import tpu as pltpu
from jax.experimental.pallas import tpu_sc as plsc

_NC = 2
_NS = 16
_NW = _NC * _NS
_L = 16
_K = 128
_DW = 16


def _sc_mesh():
    return plsc.VectorSubcoreMesh(core_axis_name="c", subcore_axis_name="s",
                                  num_cores=_NC, num_subcores=_NS)


def _deg_kernel(n_pad, nchunk, rpt):
    def body(idx_hbm, ewb_hbm, zero_hbm, out_hbm, idx_v, ewb_v, rows, acc):
        c = lax.axis_index("c")
        s = lax.axis_index("s")
        wid = c * _NS + s
        row0 = s * rpt
        pltpu.sync_copy(zero_hbm.at[pl.ds(row0, rpt)], acc.at[pl.ds(row0, rpt)])
        plsc.subcore_barrier()

        def chunk(ci, carry):
            pltpu.sync_copy(idx_hbm.at[wid, ci], idx_v)
            pltpu.sync_copy(ewb_hbm.at[wid, ci], ewb_v)

            def grp(g, carry2):
                for l in range(_L):
                    e = g * _L + l
                    w = ewb_v[e, :]
                    for j in range(128 // _L):
                        rows[e, pl.ds(j * _L, _L)] = w
                return carry2

            lax.fori_loop(0, _K // _L, grp, 0)
            pltpu.sync_copy(rows, acc.at[idx_v.at[1]], add=True)
            return carry

        lax.fori_loop(0, nchunk, chunk, 0)
        plsc.subcore_barrier()
        pltpu.sync_copy(acc.at[pl.ds(row0, rpt)],
                        out_hbm.at[c, pl.ds(row0, rpt)])

    return pl.kernel(
        body,
        out_type=jax.ShapeDtypeStruct((_NC, n_pad, 128), jnp.float32),
        mesh=_sc_mesh(),
        scratch_types=[
            pltpu.VMEM((2, _K), jnp.int32),
            pltpu.VMEM((_K, _DW), jnp.float32),
            pltpu.VMEM((_K, 128), jnp.float32),
            pltpu.VMEM_SHARED((n_pad, 128), jnp.float32),
        ],
    )


def _agg_kernel(d, n_pad, nchunk, rpt):
    def body(h_hbm, idx_hbm, ewb_hbm, zero_hbm, out_hbm,
             idx_v, ewb_v, rows, acc, sem):
        c = lax.axis_index("c")
        s = lax.axis_index("s")
        wid = c * _NS + s
        row0 = s * rpt
        pltpu.sync_copy(zero_hbm.at[pl.ds(row0, rpt)], acc.at[pl.ds(row0, rpt)])
        plsc.subcore_barrier()

        def chunk(ci, carry):
            pltpu.sync_copy(idx_hbm.at[wid, ci], idx_v)
            pltpu.sync_copy(ewb_hbm.at[wid, ci], ewb_v)
            pltpu.async_copy(h_hbm.at[idx_v.at[0]], rows, sem).wait()

            def grp(g, carry2):
                for l in range(_L):
                    e = g * _L + l
                    w = ewb_v[e, :]
                    for j in range(d // _L):
                        sl = pl.ds(j * _L, _L)
                        rows[e, sl] = rows[e, sl] * w
                return carry2

            lax.fori_loop(0, _K // _L, grp, 0)
            pltpu.sync_copy(rows, acc.at[idx_v.at[1]], add=True)
            return carry

        lax.fori_loop(0, nchunk, chunk, 0)
        plsc.subcore_barrier()
        pltpu.sync_copy(acc.at[pl.ds(row0, rpt)], out_hbm.at[c, pl.ds(row0, rpt)])

    return pl.kernel(
        body,
        out_type=jax.ShapeDtypeStruct((_NC, n_pad, d), jnp.float32),
        mesh=_sc_mesh(),
        scratch_types=[
            pltpu.VMEM((2, _K), jnp.int32),
            pltpu.VMEM((_K, _DW), jnp.float32),
            pltpu.VMEM((_K, d), jnp.float32),
            pltpu.VMEM_SHARED((n_pad, d), jnp.float32),
            pltpu.SemaphoreType.DMA,
        ],
    )


def _tc_prep(n, d):
    def body(x_ref, w_ref, dp_ref, dis_ref, hp_ref):
        deg = dp_ref[0, :n, 0:1] + dp_ref[1, :n, 0:1] + 1.0
        dis = lax.rsqrt(deg)
        dis_ref[...] = dis
        h = jnp.dot(x_ref[...], w_ref[...], preferred_element_type=jnp.float32)
        hp_ref[...] = h * dis

    return pl.pallas_call(
        body,
        out_shape=[
            jax.ShapeDtypeStruct((n, 1), jnp.float32),
            jax.ShapeDtypeStruct((n, d), jnp.float32),
        ],
    )


def _tc_mid(n, d):
    def body(ap_ref, hp_ref, dis_ref, b_ref, w_ref, out_ref):
        dis = dis_ref[...]
        pre = (ap_ref[0, :n, :] + ap_ref[1, :n, :] + hp_ref[...]) * dis + b_ref[...]
        h = jnp.maximum(pre, 0.0)
        out_ref[...] = jnp.dot(h, w_ref[...], preferred_element_type=jnp.float32) * dis

    return pl.pallas_call(
        body,
        out_shape=[jax.ShapeDtypeStruct((n, d), jnp.float32)],
    )


def _tc_head(n, d, nb, dout):
    def body(ap_ref, hp_ref, dis_ref, b_ref, batch_ref, bs_ref, u_ref,
             wha_ref, whb_ref, bh1_ref, wh2_ref, bh2_ref, out_ref):
        dis = dis_ref[...]
        h2 = jnp.maximum(
            (ap_ref[0, :n, :] + ap_ref[1, :n, :] + hp_ref[...]) * dis + b_ref[...],
            0.0)
        seg = lax.broadcasted_iota(jnp.int32, (nb, n), 0)
        onehot = (seg == batch_ref[...]).astype(jnp.float32)
        sums = jnp.dot(onehot, h2, preferred_element_type=jnp.float32)
        cnts = jnp.sum(onehot, axis=1, keepdims=True)
        gem = sums / jnp.maximum(cnts, 1.0)
        valid = lax.broadcasted_iota(jnp.int32, (nb, 1), 0) < bs_ref[0]
        gem = jnp.where(valid, gem, 0.0)
        z = jnp.maximum(
            jnp.dot(gem, wha_ref[...], preferred_element_type=jnp.float32)
            + jnp.dot(u_ref[...], whb_ref[...], preferred_element_type=jnp.float32)
            + bh1_ref[...], 0.0)
        out_ref[...] = (jnp.dot(z, wh2_ref[...], preferred_element_type=jnp.float32)
                        + bh2_ref[...])

    nargs = 12
    specs = [pl.BlockSpec(memory_space=pltpu.VMEM) for _ in range(nargs)]
    specs[5] = pl.BlockSpec(memory_space=pltpu.SMEM)
    return pl.pallas_call(
        body,
        in_specs=specs,
        out_specs=[pl.BlockSpec(memory_space=pltpu.VMEM)],
        out_shape=[jax.ShapeDtypeStruct((nb, dout), jnp.float32)],
    )


def kernel(x, edge_index, u, edge_weight, batch, batch_size,
           W1, b1, W2, b2, Wh1, bh1, Wh2, bh2):
    n, _ = x.shape
    d = W1.shape[1]
    e = edge_weight.shape[0]
    nb, dg = u.shape
    dout = Wh2.shape[1]

    per_tile = -(-e // _NW)
    nchunk = -(-per_tile // _K)
    rpt = (-(-n // _NS) + 7) // 8 * 8
    n_pad = rpt * _NS
    p = _NW * nchunk * _K

    src = edge_index[0]
    dst = edge_index[1]
    pad = p - e
    zpad_i = jnp.zeros((pad,), jnp.int32)
    srcp = jnp.concatenate([src, zpad_i]).reshape(_NW, nchunk, 1, _K)
    dstp = jnp.concatenate([dst, zpad_i]).reshape(_NW, nchunk, 1, _K)
    idxp = jnp.concatenate([srcp, dstp], axis=2)
    ewp = jnp.concatenate([edge_weight, jnp.zeros((pad,), jnp.float32)])
    ewb = jnp.broadcast_to(ewp[:, None], (p, _DW)).reshape(_NW, nchunk, _K, _DW)
    zero_w = jnp.zeros((n_pad, d), jnp.float32)

    degp = _deg_kernel(n_pad, nchunk, rpt)(idxp, ewb, zero_w)
    dis, h1p = _tc_prep(n, d)(x, W1, degp)

    agg = _agg_kernel(d, n_pad, nchunk, rpt)
    a1 = agg(h1p, idxp, ewb, zero_w)
    (h2p,) = _tc_mid(n, d)(a1, h1p, dis, b1.reshape(1, d), W2)
    a2 = agg(h2p, idxp, ewb, zero_w)

    (out,) = _tc_head(n, d, nb, dout)(
        a2, h2p, dis, b2.reshape(1, d),
        batch.reshape(1, n).astype(jnp.int32),
        jnp.asarray(batch_size, jnp.int32).reshape(1),
        u, Wh1[:d], Wh1[d:], bh1.reshape(1, d), Wh2, bh2.reshape(1, dout))
    return out
